# Initial kernel scaffold; baseline (speedup 1.0000x reference)
import jax
import jax.numpy as jnp
from jax import lax
from jax.experimental import pallas as pl
from jax.experimental.pallas import tpu as pltpu
from jax.experimental.pallas import tpu_sc as plsc

_B = 16384
_NUM_REG = 10
_NUM_CAT = 26
_VOCAB = 100000
_D = 32
_NCOLS = _NUM_REG + _NUM_CAT

_NC = 2
_NS = 16
_NW = _NC * _NS
_BPW = _B // _NW


def _body(ain_hbm, table_hbm, regw_hbm, regb_hbm, out_hbm,
          ain_v, gidx_v, rows_v, reg_v, regw_v, regb_v, sem):
    wid = lax.axis_index("s") * _NC + lax.axis_index("c")
    base = wid * _BPW

    pltpu.sync_copy(ain_hbm.at[pl.ds(base, _BPW), :], ain_v)
    pltpu.sync_copy(regw_hbm, regw_v)
    pltpu.sync_copy(regb_hbm, regb_v)

    iota = lax.iota(jnp.int32, 16)

    def build(g, _):
        row = g * 16 + iota
        for i in range(_NUM_CAT):
            col = jnp.full((16,), _NUM_REG + i, jnp.int32)
            v = plsc.load_gather(ain_v, [row, col])
            gidx_v[i, pl.ds(g * 16, 16)] = v + jnp.int32(i * _VOCAB)
        return 0
    lax.fori_loop(0, _BPW // 16, build, 0)

    def cat_step(i, _):
        pltpu.async_copy(table_hbm.at[gidx_v.at[i]], rows_v, sem).wait()
        pltpu.sync_copy(rows_v, out_hbm.at[pl.ds(base, _BPW), i])
        return 0
    lax.fori_loop(0, _NUM_CAT, cat_step, 0)

    for j in range(_NUM_REG):
        w0 = regw_v[j, pl.ds(0, 16)]
        w1 = regw_v[j, pl.ds(16, 16)]
        b0 = regb_v[j, pl.ds(0, 16)]
        b1 = regb_v[j, pl.ds(16, 16)]

        def reg_step(b, _):
            x = ain_v[b, j].astype(jnp.float32)
            reg_v[b, pl.ds(0, 16)] = x * w0 + b0
            reg_v[b, pl.ds(16, 16)] = x * w1 + b1
            return 0
        lax.fori_loop(0, _BPW, reg_step, 0)
        pltpu.sync_copy(reg_v, out_hbm.at[pl.ds(base, _BPW), _NUM_CAT + j])


_sc_call = pl.kernel(
    _body,
    out_type=jax.ShapeDtypeStruct((_B, _NCOLS, _D), jnp.float32),
    mesh=plsc.VectorSubcoreMesh(core_axis_name="c", subcore_axis_name="s"),
    scratch_types=[
        pltpu.VMEM((_BPW, _NCOLS), jnp.int32),
        pltpu.VMEM((_NUM_CAT, _BPW), jnp.int32),
        pltpu.VMEM((_BPW, _D), jnp.float32),
        pltpu.VMEM((_BPW, _D), jnp.float32),
        pltpu.VMEM((_NUM_REG, _D), jnp.float32),
        pltpu.VMEM((_NUM_REG, _D), jnp.float32),
        pltpu.SemaphoreType.DMA,
    ],
)


@jax.jit
def kernel(all_inputs, emb_tables, reg_W, reg_b):
    table = emb_tables.reshape(_NUM_CAT * _VOCAB, _D)
    return _sc_call(all_inputs, table, reg_W, reg_b)


# speedup vs baseline: 3.7027x; 3.7027x over previous
import jax
import jax.numpy as jnp
from jax import lax
from jax.experimental import pallas as pl
from jax.experimental.pallas import tpu as pltpu
from jax.experimental.pallas import tpu_sc as plsc

_B = 16384
_NUM_REG = 10
_NUM_CAT = 26
_VOCAB = 100000
_D = 32
_NCOLS = _NUM_REG + _NUM_CAT

_NC = 2
_NS = 16
_NW = _NC * _NS
_BC = 8192


def _body(ainT_hbm, tableT_hbm, regw_hbm, regb_hbm, outT_hbm,
          row_v, idx_v, out_v, regw_v, regb_v):
    w = lax.axis_index("s") * _NC + lax.axis_index("c")

    pltpu.sync_copy(regw_hbm, regw_v)
    pltpu.sync_copy(regb_hbm, regb_v)

    for i in range(_NUM_CAT):
        pltpu.sync_copy(tableT_hbm.at[i, w], row_v)
        for h in range(_B // _BC):
            pltpu.sync_copy(ainT_hbm.at[_NUM_REG + i, pl.ds(h * _BC, _BC)],
                            idx_v)

            def gstep(g, _):
                iv = idx_v[pl.ds(g * 16, 16)]
                out_v[pl.ds(g * 16, 16)] = plsc.load_gather(row_v, [iv])
                return 0
            lax.fori_loop(0, _BC // 16, gstep, 0)
            pltpu.sync_copy(out_v, outT_hbm.at[i, w, pl.ds(h * _BC, _BC)])

    for j in range(_NUM_REG):
        sel = jnp.full((16,), j * _D, jnp.int32) + w
        ws = plsc.load_gather(regw_v, [sel])
        bs = plsc.load_gather(regb_v, [sel])
        for h in range(_B // _BC):
            pltpu.sync_copy(ainT_hbm.at[j, pl.ds(h * _BC, _BC)], idx_v)

            def rstep(g, _):
                xf = idx_v[pl.ds(g * 16, 16)].astype(jnp.float32)
                out_v[pl.ds(g * 16, 16)] = xf * ws + bs
                return 0
            lax.fori_loop(0, _BC // 16, rstep, 0)
            pltpu.sync_copy(out_v, outT_hbm.at[_NUM_CAT + j, w,
                                               pl.ds(h * _BC, _BC)])


_sc_call = pl.kernel(
    _body,
    out_type=jax.ShapeDtypeStruct((_NCOLS, _D, _B), jnp.float32),
    mesh=plsc.VectorSubcoreMesh(core_axis_name="c", subcore_axis_name="s"),
    scratch_types=[
        pltpu.VMEM((_VOCAB,), jnp.float32),
        pltpu.VMEM((_BC,), jnp.int32),
        pltpu.VMEM((_BC,), jnp.float32),
        pltpu.VMEM((_NUM_REG * _D,), jnp.float32),
        pltpu.VMEM((_NUM_REG * _D,), jnp.float32),
    ],
    compiler_params=pltpu.CompilerParams(
        needs_layout_passes=False, use_tc_tiling_on_sc=True),
)


@jax.jit
def kernel(all_inputs, emb_tables, reg_W, reg_b):
    ainT = all_inputs.T
    tableT = emb_tables.transpose(0, 2, 1)
    outT = _sc_call(ainT, tableT, reg_W.reshape(-1), reg_b.reshape(-1))
    return outT.transpose(2, 0, 1)
